# baseline (speedup 1.0000x reference)
import functools

import jax
import jax.numpy as jnp
from jax import lax
from jax.experimental import pallas as pl
from jax.experimental.pallas import tpu as pltpu
from jax.experimental.pallas import tpu_sc as plsc

_N_STATES = 256
_N_OBVS = 100000
_BATCH = 16384

_NC = 2
_NS = 16
_NW = _NC * _NS
_LANES = 16

_CHUNK = 128
_ROWS_PER_W = _BATCH // (_NW * _CHUNK)
_ELEMS_PER_W = _ROWS_PER_W * _CHUNK

_LROWS = _N_STATES * _N_OBVS // _CHUNK

_N_STRIPS = 8
_STRIP_ROWS = _N_STATES // _N_STRIPS
_STRIP_LROWS = _STRIP_ROWS * _N_OBVS // _CHUNK


def _emission_gather(lrows_mat, state2d, obs2d):
    mesh = plsc.VectorSubcoreMesh(core_axis_name="c", subcore_axis_name="s")

    @functools.partial(
        pl.kernel,
        mesh=mesh,
        compiler_params=pltpu.CompilerParams(use_tc_tiling_on_sc=False),
        out_type=jax.ShapeDtypeStruct((_BATCH // _CHUNK, _CHUNK), jnp.float32),
        scratch_types=[
            pltpu.VMEM((_ROWS_PER_W, _CHUNK), jnp.int32),
            pltpu.VMEM((_ROWS_PER_W, _CHUNK), jnp.int32),
            pltpu.VMEM((_ROWS_PER_W, _CHUNK), jnp.int32),
            pltpu.VMEM((_ROWS_PER_W, _CHUNK), jnp.int32),
            pltpu.VMEM((_ELEMS_PER_W, _CHUNK), jnp.float32),
            pltpu.VMEM((_ROWS_PER_W, _CHUNK), jnp.float32),
            pltpu.VMEM((_LANES,), jnp.int32),
            pltpu.VMEM((_ROWS_PER_W, _CHUNK), jnp.int32),
            pltpu.VMEM_SHARED((_NS * _ELEMS_PER_W // 2, _CHUNK), jnp.float32),
            pltpu.SemaphoreType.DMA,
        ],
    )
    def k(table_hbm, state_hbm, obs_hbm, ramp_hbm, out_hbm,
          st_v, ob_v, ridx_v, lane_v, rows_v, val_v, ramp_v, off_v, shr, sem):
        wid = lax.axis_index("s") * _NC + lax.axis_index("c")
        base = wid * _ROWS_PER_W
        pltpu.sync_copy(state_hbm.at[pl.ds(base, _ROWS_PER_W)], st_v)
        pltpu.sync_copy(obs_hbm.at[pl.ds(base, _ROWS_PER_W)], ob_v)
        pltpu.sync_copy(ramp_hbm, ramp_v)
        for j in range(_ROWS_PER_W):
            for t in range(_CHUNK // _LANES):
                sl = pl.ds(t * _LANES, _LANES)
                flat = st_v[j, sl] * _N_OBVS + ob_v[j, sl]
                ridx_v[j, sl] = flat >> 7
                lane_v[j, sl] = flat & 127
        copies = [
            pltpu.async_copy(
                table_hbm.at[ridx_v.at[j]],
                rows_v.at[pl.ds(j * _CHUNK, _CHUNK)],
                sem,
            )
            for j in range(_ROWS_PER_W)
        ]
        for c in copies:
            c.wait()
        half_elems = _ELEMS_PER_W // 2
        half_rows = _ROWS_PER_W // 2
        srow = lax.axis_index("s") * half_elems
        lane16 = ramp_v[...]
        sbase = srow * _CHUNK
        for h in range(2):
            pltpu.sync_copy(
                rows_v.at[pl.ds(h * half_elems, half_elems)],
                shr.at[pl.ds(srow, half_elems)],
            )
            for jj in range(half_rows):
                j = h * half_rows + jj
                for t in range(_CHUNK // _LANES):
                    sl = pl.ds(t * _LANES, _LANES)
                    elem = lane16 + (jj * _CHUNK + t * _LANES)
                    off_v[j, sl] = sbase + elem * _CHUNK + lane_v[j, sl]
            picks = [
                pltpu.async_copy(
                    shr.at[0].at[off_v.at[h * half_rows + jj]],
                    val_v.at[h * half_rows + jj],
                    sem,
                )
                for jj in range(half_rows)
            ]
            for p in picks:
                p.wait()
        pltpu.sync_copy(val_v, out_hbm.at[pl.ds(base, _ROWS_PER_W)])

    return k(lrows_mat, state2d, obs2d, jnp.arange(_LANES, dtype=jnp.int32))


def kernel(log_em, state, obs):
    lrows_mat = log_em.reshape(_LROWS, _CHUNK)
    state2d = state.reshape(_BATCH // _CHUNK, _CHUNK)
    obs2d = obs.reshape(_BATCH // _CHUNK, _CHUNK)
    out2d = _emission_gather(lrows_mat, state2d, obs2d)
    return out2d.reshape(-1)


# speedup vs baseline: 1.0229x; 1.0229x over previous
import functools

import jax
import jax.numpy as jnp
from jax import lax
from jax.experimental import pallas as pl
from jax.experimental.pallas import tpu as pltpu
from jax.experimental.pallas import tpu_sc as plsc

_N_STATES = 256
_N_OBVS = 100000
_BATCH = 16384

_NC = 2
_NS = 16
_NW = _NC * _NS
_LANES = 16

_CHUNK = 128
_ROWS_PER_W = _BATCH // (_NW * _CHUNK)


def _emission_gather(table_flat, state2d, obs2d):
    mesh = plsc.VectorSubcoreMesh(core_axis_name="c", subcore_axis_name="s")

    @functools.partial(
        pl.kernel,
        mesh=mesh,
        compiler_params=pltpu.CompilerParams(use_tc_tiling_on_sc=False),
        out_type=jax.ShapeDtypeStruct((_BATCH // _CHUNK, _CHUNK), jnp.float32),
        scratch_types=[
            pltpu.VMEM((_ROWS_PER_W, _CHUNK), jnp.int32),
            pltpu.VMEM((_ROWS_PER_W, _CHUNK), jnp.int32),
            pltpu.VMEM((_ROWS_PER_W, _CHUNK), jnp.int32),
            pltpu.VMEM((_ROWS_PER_W, _CHUNK), jnp.float32),
            pltpu.SemaphoreType.DMA,
        ],
    )
    def k(table_hbm, state_hbm, obs_hbm, out_hbm, st_v, ob_v, idx_v, val_v, sem):
        wid = lax.axis_index("s") * _NC + lax.axis_index("c")
        base = wid * _ROWS_PER_W
        pltpu.sync_copy(state_hbm.at[pl.ds(base, _ROWS_PER_W)], st_v)
        pltpu.sync_copy(obs_hbm.at[pl.ds(base, _ROWS_PER_W)], ob_v)
        for j in range(_ROWS_PER_W):
            for t in range(_CHUNK // _LANES):
                sl = pl.ds(t * _LANES, _LANES)
                idx_v[j, sl] = st_v[j, sl] * _N_OBVS + ob_v[j, sl]
        copies = [
            pltpu.async_copy(table_hbm.at[idx_v.at[j]], val_v.at[j], sem)
            for j in range(_ROWS_PER_W)
        ]
        for c in copies:
            c.wait()
        pltpu.sync_copy(val_v, out_hbm.at[pl.ds(base, _ROWS_PER_W)])

    return k(table_flat, state2d, obs2d)


def kernel(log_em, state, obs):
    table_flat = log_em.reshape(-1)
    state2d = state.reshape(_BATCH // _CHUNK, _CHUNK)
    obs2d = obs.reshape(_BATCH // _CHUNK, _CHUNK)
    out2d = _emission_gather(table_flat, state2d, obs2d)
    return out2d.reshape(-1)
